# Initial kernel scaffold; baseline (speedup 1.0000x reference)
import jax
import jax.numpy as jnp
from jax.experimental import pallas as pl


def kernel(gt_ctrs, ctrness_pred, rand_vals, unique_id):
    raise NotImplementedError("write your pallas kernel here")


# speedup vs baseline: 1.7680x; 1.7680x over previous
import functools

import jax
import jax.numpy as jnp
from jax.experimental import pallas as pl
from jax.experimental.pallas import tpu as pltpu

_LANES = 128
_ROWS_PAD = 16384
_N_PAD = _ROWS_PAD * _LANES
_BLK_ROWS = 128
_THETA = 0.02


def _loss_body(n_real, a_gt_ref, b_gt_ref, a_pr_ref, b_pr_ref, out_ref,
               acc_ref):
    i = pl.program_id(0)

    @pl.when(i == 0)
    def _init():
        acc_ref[0] = 0.0
        acc_ref[1] = 0.0
        acc_ref[2] = 0.0

    za = a_gt_ref[...]
    zb = b_gt_ref[...]
    flag1 = za / zb
    flag2 = zb / za
    mask1 = flag1 > 1.0 + _THETA
    mask2 = flag2 > 1.0 + _THETA
    target = jnp.where(mask1, 1.0, 0.0)
    target = jnp.where(mask2, -1.0, target)

    z_a = jax.nn.sigmoid(a_pr_ref[...])
    z_b = jax.nn.sigmoid(b_pr_ref[...])
    d = z_a - z_b
    nz = target != 0.0
    log_terms = jnp.log(1.0 + jnp.exp(-target * d))

    acc_ref[0] = acc_ref[0] + jnp.sum(jnp.where(nz, log_terms, 0.0))
    acc_ref[1] = acc_ref[1] + jnp.sum(jnp.where(nz, 0.0, d * d))
    acc_ref[2] = acc_ref[2] + jnp.sum(nz.astype(jnp.float32))

    @pl.when(i == pl.num_programs(0) - 1)
    def _fini():
        n_nz = jnp.maximum(acc_ref[2], 1.0)
        n_z = jnp.maximum(jnp.float32(n_real) - acc_ref[2], 1.0)
        out_ref[0, 0] = acc_ref[0] / n_nz + acc_ref[1] / n_z


def _loss_pallas(a_gt, b_gt, a_pr, b_pr, n_real):
    rows = a_gt.shape[0] // _LANES
    grid = rows // _BLK_ROWS
    spec = pl.BlockSpec((_BLK_ROWS, _LANES), lambda i: (i, 0))
    out = pl.pallas_call(
        functools.partial(_loss_body, n_real),
        grid=(grid,),
        in_specs=[spec, spec, spec, spec],
        out_specs=pl.BlockSpec(memory_space=pltpu.SMEM),
        out_shape=jax.ShapeDtypeStruct((1, 1), jnp.float32),
        scratch_shapes=[pltpu.SMEM((4,), jnp.float32)],
    )(a_gt.reshape(rows, _LANES), b_gt.reshape(rows, _LANES),
      a_pr.reshape(rows, _LANES), b_pr.reshape(rows, _LANES))
    return out[0, 0]


def kernel(gt_ctrs, ctrness_pred, rand_vals, unique_id):
    n = gt_ctrs.shape[0]
    order_rand = jnp.lexsort((rand_vals, unique_id))
    order_stable = jnp.lexsort((jnp.arange(n), unique_id))

    pad = _N_PAD - n
    t = jnp.concatenate([order_stable, jnp.zeros((pad,), order_stable.dtype)])
    s = jnp.concatenate([order_rand, jnp.zeros((pad,), order_rand.dtype)])

    a_gt = jnp.take(gt_ctrs, t, axis=0)
    b_gt = jnp.take(gt_ctrs, s, axis=0)
    a_pr = jnp.take(ctrness_pred, t, axis=0)
    b_pr = jnp.take(ctrness_pred, s, axis=0)
    return _loss_pallas(a_gt, b_gt, a_pr, b_pr, n)


# speedup vs baseline: 1.9716x; 1.1152x over previous
import functools

import jax
import jax.numpy as jnp
from jax import lax
from jax.experimental import pallas as pl
from jax.experimental.pallas import tpu as pltpu
from jax.experimental.pallas import tpu_sc as plsc

_LANES = 128
_N_PAD = 1 << 21
_ROWS = _N_PAD // _LANES
_NW = 32
_WROWS = _ROWS // _NW
_SLAB_ROWS = 32
_SLABS = _WROWS // _SLAB_ROWS
_BLK_ROWS = 128
_THETA = 0.02

_mesh = plsc.VectorSubcoreMesh(core_axis_name="c", subcore_axis_name="s")


def _gather_body(t2d, s2d, gtflat, prflat,
                 a_gt2d, b_gt2d, a_pr2d, b_pr2d,
                 tbuf, sbuf, agbuf, bgbuf, apbuf, bpbuf, sem):
    w = lax.axis_index("s") * 2 + lax.axis_index("c")

    @pl.loop(0, _SLABS)
    def _slab(sl):
        r0 = w * _WROWS + sl * _SLAB_ROWS
        pltpu.sync_copy(t2d.at[pl.ds(r0, _SLAB_ROWS)], tbuf)
        pltpu.sync_copy(s2d.at[pl.ds(r0, _SLAB_ROWS)], sbuf)

        @pl.loop(0, _SLAB_ROWS)
        def _fire(r):
            pltpu.async_copy(gtflat.at[tbuf.at[r]], agbuf.at[r], sem)
            pltpu.async_copy(gtflat.at[sbuf.at[r]], bgbuf.at[r], sem)
            pltpu.async_copy(prflat.at[tbuf.at[r]], apbuf.at[r], sem)
            pltpu.async_copy(prflat.at[sbuf.at[r]], bpbuf.at[r], sem)

        for buf in (agbuf, bgbuf, apbuf, bpbuf):
            pltpu.make_async_copy(a_gt2d.at[pl.ds(r0, _SLAB_ROWS)], buf,
                                  sem).wait()
        pltpu.sync_copy(agbuf, a_gt2d.at[pl.ds(r0, _SLAB_ROWS)])
        pltpu.sync_copy(bgbuf, b_gt2d.at[pl.ds(r0, _SLAB_ROWS)])
        pltpu.sync_copy(apbuf, a_pr2d.at[pl.ds(r0, _SLAB_ROWS)])
        pltpu.sync_copy(bpbuf, b_pr2d.at[pl.ds(r0, _SLAB_ROWS)])


_gather_call = pl.kernel(
    _gather_body, mesh=_mesh,
    out_type=(jax.ShapeDtypeStruct((_ROWS, _LANES), jnp.float32),
              jax.ShapeDtypeStruct((_ROWS, _LANES), jnp.float32),
              jax.ShapeDtypeStruct((_ROWS, _LANES), jnp.float32),
              jax.ShapeDtypeStruct((_ROWS, _LANES), jnp.float32)),
    scratch_types=[pltpu.VMEM((_SLAB_ROWS, _LANES), jnp.int32),
                   pltpu.VMEM((_SLAB_ROWS, _LANES), jnp.int32),
                   pltpu.VMEM((_SLAB_ROWS, _LANES), jnp.float32),
                   pltpu.VMEM((_SLAB_ROWS, _LANES), jnp.float32),
                   pltpu.VMEM((_SLAB_ROWS, _LANES), jnp.float32),
                   pltpu.VMEM((_SLAB_ROWS, _LANES), jnp.float32),
                   pltpu.SemaphoreType.DMA])


def _loss_body(n_real, a_gt_ref, b_gt_ref, a_pr_ref, b_pr_ref, out_ref,
               acc_ref):
    i = pl.program_id(0)

    @pl.when(i == 0)
    def _init():
        acc_ref[0] = 0.0
        acc_ref[1] = 0.0
        acc_ref[2] = 0.0

    za = a_gt_ref[...]
    zb = b_gt_ref[...]
    mask1 = za / zb > 1.0 + _THETA
    mask2 = zb / za > 1.0 + _THETA
    target = jnp.where(mask1, 1.0, 0.0)
    target = jnp.where(mask2, -1.0, target)

    z_a = jax.nn.sigmoid(a_pr_ref[...])
    z_b = jax.nn.sigmoid(b_pr_ref[...])
    d = z_a - z_b
    nz = target != 0.0
    log_terms = jnp.log(1.0 + jnp.exp(-target * d))

    acc_ref[0] = acc_ref[0] + jnp.sum(jnp.where(nz, log_terms, 0.0))
    acc_ref[1] = acc_ref[1] + jnp.sum(jnp.where(nz, 0.0, d * d))
    acc_ref[2] = acc_ref[2] + jnp.sum(nz.astype(jnp.float32))

    @pl.when(i == pl.num_programs(0) - 1)
    def _fini():
        n_nz = jnp.maximum(acc_ref[2], 1.0)
        n_z = jnp.maximum(jnp.float32(n_real) - acc_ref[2], 1.0)
        out_ref[0, 0] = acc_ref[0] / n_nz + acc_ref[1] / n_z


def _loss_pallas(a_gt, b_gt, a_pr, b_pr, n_real):
    spec = pl.BlockSpec((_BLK_ROWS, _LANES), lambda i: (i, 0))
    out = pl.pallas_call(
        functools.partial(_loss_body, n_real),
        grid=(_ROWS // _BLK_ROWS,),
        in_specs=[spec, spec, spec, spec],
        out_specs=pl.BlockSpec(memory_space=pltpu.SMEM),
        out_shape=jax.ShapeDtypeStruct((1, 1), jnp.float32),
        scratch_shapes=[pltpu.SMEM((4,), jnp.float32)],
    )(a_gt, b_gt, a_pr, b_pr)
    return out[0, 0]


def kernel(gt_ctrs, ctrness_pred, rand_vals, unique_id):
    n = gt_ctrs.shape[0]
    pad = _N_PAD - n

    order_rand = jnp.lexsort((rand_vals, unique_id))
    order_stable = jnp.lexsort((jnp.arange(n), unique_id))

    t = jnp.concatenate(
        [order_stable.astype(jnp.int32), jnp.zeros((pad,), jnp.int32)])
    s = jnp.concatenate(
        [order_rand.astype(jnp.int32), jnp.zeros((pad,), jnp.int32)])

    a_gt, b_gt, a_pr, b_pr = _gather_call(
        t.reshape(_ROWS, _LANES), s.reshape(_ROWS, _LANES),
        gt_ctrs, ctrness_pred)
    return _loss_pallas(a_gt, b_gt, a_pr, b_pr, n)


# speedup vs baseline: 1.9881x; 1.0084x over previous
import functools

import jax
import jax.numpy as jnp
from jax import lax
from jax.experimental import pallas as pl
from jax.experimental.pallas import tpu as pltpu
from jax.experimental.pallas import tpu_sc as plsc

_LANES = 128
_N_PAD = 1 << 21
_ROWS = _N_PAD // _LANES
_NW = 32
_CHUNK = _N_PAD // _NW
_WROWS = _CHUNK // _LANES
_LPW = _CHUNK // 16
_NGROUPS = 4096
_NB = 4112
_HSZ = 16 * _NB
_SLAB_ROWS = 32
_SLABS = _WROWS // _SLAB_ROWS
_BLK_ROWS = 128
_THETA = 0.02

_mesh = plsc.VectorSubcoreMesh(core_axis_name="c", subcore_axis_name="s")
_sc_params = pltpu.CompilerParams(needs_layout_passes=False)


def _wid():
    return lax.axis_index("s") * 2 + lax.axis_index("c")


def _hist_body(uid2d, hist_out, tot_out, hist, ubuf, totv):
    w = _wid()
    lane_off = lax.iota(jnp.int32, 16) * _NB
    ones = jnp.ones((16,), jnp.int32)
    zeros = jnp.zeros((16,), jnp.int32)

    @pl.loop(0, _HSZ // 16)
    def _zero(i):
        hist[pl.ds(i * 16, 16)] = zeros

    @pl.loop(0, _SLABS)
    def _slab(sl):
        r0 = w * _WROWS + sl * _SLAB_ROWS
        pltpu.sync_copy(uid2d.at[pl.ds(r0, _SLAB_ROWS)], ubuf)

        @pl.loop(0, _SLAB_ROWS)
        def _row(r):
            for cc in range(8):
                v = ubuf[r, pl.ds(cc * 16, 16)]
                plsc.addupdate_scatter(hist, [lane_off + v], ones)

    pltpu.sync_copy(hist, hist_out.at[w])

    @pl.loop(0, _NB // 16)
    def _tot(j):
        acc = zeros
        for l in range(16):
            acc = acc + hist[pl.ds(l * _NB + j * 16, 16)]
        totv[pl.ds(j * 16, 16)] = acc

    pltpu.sync_copy(totv, tot_out.at[w])


_hist_call = pl.kernel(
    _hist_body, mesh=_mesh, compiler_params=_sc_params,
    out_type=(jax.ShapeDtypeStruct((_NW, _HSZ), jnp.int32),
              jax.ShapeDtypeStruct((_NW, _NB), jnp.int32)),
    scratch_types=[pltpu.VMEM((_HSZ,), jnp.int32),
                   pltpu.VMEM((_SLAB_ROWS, _LANES), jnp.int32),
                   pltpu.VMEM((_NB,), jnp.int32)])


def _rank_body(uid2d, gtp2d, prp2d, s2d, gtflat, prflat, hist_hbm, tot_hbm,
               a_gt, a_pr, b_gt2d, b_pr2d,
               hist, tbuf, gtot, pre, ubuf, sbuf, gbuf, pbuf, rbuf,
               bgbuf, bpbuf, sem):
    w = _wid()
    lane_off = lax.iota(jnp.int32, 16) * _NB
    zeros = jnp.zeros((16,), jnp.int32)

    pltpu.sync_copy(hist_hbm.at[w], hist)

    @pl.loop(0, _NB // 16)
    def _zero(j):
        gtot[pl.ds(j * 16, 16)] = zeros
        pre[pl.ds(j * 16, 16)] = zeros

    for w2 in range(_NW):
        pltpu.sync_copy(tot_hbm.at[w2], tbuf)
        m = (jnp.int32(w2) < w).astype(jnp.int32)

        @pl.loop(0, _NB // 16)
        def _acc(j, _m=m):
            t = tbuf[pl.ds(j * 16, 16)]
            gtot[pl.ds(j * 16, 16)] = gtot[pl.ds(j * 16, 16)] + t
            pre[pl.ds(j * 16, 16)] = pre[pl.ds(j * 16, 16)] + t * _m

    @pl.loop(0, _NB // 16, init_carry=jnp.int32(0))
    def _scan(j, carry):
        v = gtot[pl.ds(j * 16, 16)]
        ex = lax.cumsum(v, axis=0) - v + carry
        tbuf[pl.ds(j * 16, 16)] = ex + pre[pl.ds(j * 16, 16)]
        return carry + jnp.sum(v)

    @pl.loop(0, _NB // 16)
    def _base(j):
        sp = tbuf[pl.ds(j * 16, 16)]
        acc = zeros
        for l in range(16):
            h = hist[pl.ds(l * _NB + j * 16, 16)]
            hist[pl.ds(l * _NB + j * 16, 16)] = sp + acc
            acc = acc + h

    @pl.loop(0, _SLABS)
    def _slab(sl):
        r0 = w * _WROWS + sl * _SLAB_ROWS
        pltpu.sync_copy(uid2d.at[pl.ds(r0, _SLAB_ROWS)], ubuf)
        pltpu.sync_copy(gtp2d.at[pl.ds(r0, _SLAB_ROWS)], gbuf)
        pltpu.sync_copy(prp2d.at[pl.ds(r0, _SLAB_ROWS)], pbuf)
        pltpu.sync_copy(s2d.at[pl.ds(r0, _SLAB_ROWS)], sbuf)

        @pl.loop(0, _SLAB_ROWS)
        def _rank(r):
            for cc in range(8):
                v = ubuf[r, pl.ds(cc * 16, 16)]
                addr = lane_off + v
                b = plsc.load_gather(hist, [addr])
                plsc.store_scatter(hist, [addr], b + 1)
                rbuf[r, pl.ds(cc * 16, 16)] = b

        @pl.loop(0, _SLAB_ROWS)
        def _fire(r):
            pltpu.async_copy(gbuf.at[r], a_gt.at[rbuf.at[r]], sem)
            pltpu.async_copy(pbuf.at[r], a_pr.at[rbuf.at[r]], sem)
            pltpu.async_copy(gtflat.at[sbuf.at[r]], bgbuf.at[r], sem)
            pltpu.async_copy(prflat.at[sbuf.at[r]], bpbuf.at[r], sem)

        for buf in (gbuf, pbuf, bgbuf, bpbuf):
            pltpu.make_async_copy(gtp2d.at[pl.ds(r0, _SLAB_ROWS)], buf,
                                  sem).wait()

        pltpu.sync_copy(bgbuf, b_gt2d.at[pl.ds(r0, _SLAB_ROWS)])
        pltpu.sync_copy(bpbuf, b_pr2d.at[pl.ds(r0, _SLAB_ROWS)])


_rank_call = pl.kernel(
    _rank_body, mesh=_mesh, compiler_params=_sc_params,
    out_type=(jax.ShapeDtypeStruct((_N_PAD,), jnp.float32),
              jax.ShapeDtypeStruct((_N_PAD,), jnp.float32),
              jax.ShapeDtypeStruct((_ROWS, _LANES), jnp.float32),
              jax.ShapeDtypeStruct((_ROWS, _LANES), jnp.float32)),
    scratch_types=[pltpu.VMEM((_HSZ,), jnp.int32),
                   pltpu.VMEM((_NB,), jnp.int32),
                   pltpu.VMEM((_NB,), jnp.int32),
                   pltpu.VMEM((_NB,), jnp.int32),
                   pltpu.VMEM((_SLAB_ROWS, _LANES), jnp.int32),
                   pltpu.VMEM((_SLAB_ROWS, _LANES), jnp.int32),
                   pltpu.VMEM((_SLAB_ROWS, _LANES), jnp.float32),
                   pltpu.VMEM((_SLAB_ROWS, _LANES), jnp.float32),
                   pltpu.VMEM((_SLAB_ROWS, _LANES), jnp.int32),
                   pltpu.VMEM((_SLAB_ROWS, _LANES), jnp.float32),
                   pltpu.VMEM((_SLAB_ROWS, _LANES), jnp.float32),
                   pltpu.SemaphoreType.DMA])


def _loss_body(n_real, a_gt_ref, b_gt_ref, a_pr_ref, b_pr_ref, out_ref,
               acc_ref):
    i = pl.program_id(0)

    @pl.when(i == 0)
    def _init():
        acc_ref[0] = 0.0
        acc_ref[1] = 0.0
        acc_ref[2] = 0.0

    za = a_gt_ref[...]
    zb = b_gt_ref[...]
    mask1 = za / zb > 1.0 + _THETA
    mask2 = zb / za > 1.0 + _THETA
    target = jnp.where(mask1, 1.0, 0.0)
    target = jnp.where(mask2, -1.0, target)

    z_a = jax.nn.sigmoid(a_pr_ref[...])
    z_b = jax.nn.sigmoid(b_pr_ref[...])
    d = z_a - z_b
    nz = target != 0.0
    log_terms = jnp.log(1.0 + jnp.exp(-target * d))

    acc_ref[0] = acc_ref[0] + jnp.sum(jnp.where(nz, log_terms, 0.0))
    acc_ref[1] = acc_ref[1] + jnp.sum(jnp.where(nz, 0.0, d * d))
    acc_ref[2] = acc_ref[2] + jnp.sum(nz.astype(jnp.float32))

    @pl.when(i == pl.num_programs(0) - 1)
    def _fini():
        n_nz = jnp.maximum(acc_ref[2], 1.0)
        n_z = jnp.maximum(jnp.float32(n_real) - acc_ref[2], 1.0)
        out_ref[0, 0] = acc_ref[0] / n_nz + acc_ref[1] / n_z


def _loss_pallas(a_gt, b_gt, a_pr, b_pr, n_real):
    spec = pl.BlockSpec((_BLK_ROWS, _LANES), lambda i: (i, 0))
    out = pl.pallas_call(
        functools.partial(_loss_body, n_real),
        grid=(_ROWS // _BLK_ROWS,),
        in_specs=[spec, spec, spec, spec],
        out_specs=pl.BlockSpec(memory_space=pltpu.SMEM),
        out_shape=jax.ShapeDtypeStruct((1, 1), jnp.float32),
        scratch_shapes=[pltpu.SMEM((4,), jnp.float32)],
    )(a_gt.reshape(_ROWS, _LANES), b_gt.reshape(_ROWS, _LANES),
      a_pr.reshape(_ROWS, _LANES), b_pr.reshape(_ROWS, _LANES))
    return out[0, 0]


def _lane_perm(x):
    return x.reshape(_NW, 16, _LPW).transpose(0, 2, 1).reshape(_ROWS, _LANES)


def kernel(gt_ctrs, ctrness_pred, rand_vals, unique_id):
    n = gt_ctrs.shape[0]
    pad = _N_PAD - n

    order_rand = jnp.lexsort((rand_vals, unique_id))

    uid_ext = jnp.concatenate(
        [unique_id, jnp.full((pad,), _NGROUPS, jnp.int32)])
    gt_ext = jnp.concatenate([gt_ctrs, jnp.ones((pad,), jnp.float32)])
    pr_ext = jnp.concatenate([ctrness_pred, jnp.zeros((pad,), jnp.float32)])
    s_ext = jnp.concatenate(
        [order_rand.astype(jnp.int32),
         jnp.arange(n, _N_PAD, dtype=jnp.int32)])

    hist_tab, tot_tab = _hist_call(_lane_perm(uid_ext))
    a_gt, a_pr, b_gt, b_pr = _rank_call(
        _lane_perm(uid_ext), _lane_perm(gt_ext), _lane_perm(pr_ext),
        s_ext.reshape(_ROWS, _LANES), gt_ext, pr_ext, hist_tab, tot_tab)

    return _loss_pallas(a_gt, b_gt.reshape(-1), a_pr, b_pr.reshape(-1), n)
